# baseline (speedup 1.0000x reference)
import functools

import jax
import jax.numpy as jnp
from jax import lax
from jax.experimental import pallas as pl
from jax.experimental.pallas import tpu as pltpu
from jax.experimental.pallas import tpu_sc as plsc

N = 10000
F = 128
E = 320000
NC = 2
NS = 16
NW = NC * NS
L = 16
N_PAD = 10240
RPT = N_PAD // NS
EPT = 10240
E_PAD = EPT * NW
KCH = EPT // 128
ER = E_PAD // 128
ER4 = E_PAD // 64
KC4 = EPT // 64


def _perm_cast(h):
    n = h.shape[0]
    hp = h.reshape(n, 4, 2, 16).transpose(0, 1, 3, 2).reshape(n, F)
    return hp.astype(jnp.bfloat16)


def _mm_alpha_body(x_ref, w_ref, a2_ref, h_ref, aout_ref):
    h = jnp.dot(x_ref[...], w_ref[...], preferred_element_type=jnp.float32)
    h_ref[...] = _perm_cast(h)
    aout_ref[...] = jnp.dot(h, a2_ref[...], preferred_element_type=jnp.float32)


def _mm_alpha(x, w, a2, blk=1000):
    n = x.shape[0]
    return pl.pallas_call(
        _mm_alpha_body,
        grid=(n // blk,),
        in_specs=[pl.BlockSpec((blk, F), lambda i: (i, 0)),
                  pl.BlockSpec((F, F), lambda i: (0, 0)),
                  pl.BlockSpec((F, 2), lambda i: (0, 0))],
        out_specs=[pl.BlockSpec((blk, F), lambda i: (i, 0)),
                   pl.BlockSpec((blk, 2), lambda i: (i, 0))],
        out_shape=[jax.ShapeDtypeStruct((n, F), jnp.bfloat16),
                   jax.ShapeDtypeStruct((n, 2), jnp.float32)],
    )(x, w, a2)


def _mm_relu_alpha_body(p_ref, w_ref, a2_ref, h_ref, aout_ref):
    g = jnp.maximum(p_ref[0] + p_ref[1], 0.0)
    h = jnp.dot(g, w_ref[...], preferred_element_type=jnp.float32)
    h_ref[...] = _perm_cast(h)
    aout_ref[...] = jnp.dot(h, a2_ref[...], preferred_element_type=jnp.float32)


def _mm_relu_alpha(p, w, a2, blk=1024):
    n = p.shape[1]
    return pl.pallas_call(
        _mm_relu_alpha_body,
        grid=(n // blk,),
        in_specs=[pl.BlockSpec((2, blk, F), lambda i: (0, i, 0)),
                  pl.BlockSpec((F, F), lambda i: (0, 0)),
                  pl.BlockSpec((F, 2), lambda i: (0, 0))],
        out_specs=[pl.BlockSpec((blk, F), lambda i: (i, 0)),
                   pl.BlockSpec((blk, 2), lambda i: (i, 0))],
        out_shape=[jax.ShapeDtypeStruct((n, F), jnp.bfloat16),
                   jax.ShapeDtypeStruct((n, 2), jnp.float32)],
    )(p, w, a2)


def _combine_body(p_ref, o_ref):
    o_ref[...] = p_ref[0] + p_ref[1]


def _combine(p, blk=2000):
    return pl.pallas_call(
        _combine_body,
        grid=(N // blk,),
        in_specs=[pl.BlockSpec((2, blk, F), lambda i: (0, i, 0))],
        out_specs=pl.BlockSpec((blk, F), lambda i: (i, 0)),
        out_shape=jax.ShapeDtypeStruct((N, F), jnp.float32),
    )(p)


_MESH = plsc.VectorSubcoreMesh(core_axis_name="c", subcore_axis_name="s")


@functools.partial(
    pl.kernel,
    out_type=[jax.ShapeDtypeStruct((ER, 128), jnp.float32),
              jax.ShapeDtypeStruct((NC, N_PAD), jnp.float32)],
    mesh=_MESH,
    compiler_params=pltpu.CompilerParams(needs_layout_passes=False),
    scratch_types=[
        pltpu.VMEM((N_PAD,), jnp.float32),
        pltpu.VMEM((N_PAD,), jnp.float32),
        pltpu.VMEM((KCH, 128), jnp.int32),
        pltpu.VMEM((KCH, 128), jnp.int32),
        pltpu.VMEM((KCH, 128), jnp.float32),
        pltpu.VMEM((N_PAD,), jnp.float32),
        pltpu.VMEM((RPT,), jnp.float32),
        pltpu.VMEM((RPT,), jnp.float32),
        pltpu.VMEM_SHARED((NS, N_PAD), jnp.float32),
    ],
)
def _edge_softmax(srcR, dstR, asrc, adst, exR, denp,
                  asv, adv, src2, dst2, ex2, den_v, tmp_v, accv, stage):
    c = lax.axis_index("c")
    s = lax.axis_index("s")
    wid = c * NS + s
    z16 = jnp.zeros((L,), jnp.float32)

    def zset(i, _):
        asv[pl.ds(i * L, L)] = z16
        adv[pl.ds(i * L, L)] = z16
        den_v[pl.ds(i * L, L)] = z16
        return 0
    lax.fori_loop(0, N_PAD // L, zset, 0)

    pltpu.sync_copy(asrc, asv.at[pl.ds(0, N)])
    pltpu.sync_copy(adst, adv.at[pl.ds(0, N)])
    pltpu.sync_copy(srcR.at[pl.ds(wid * KCH, KCH)], src2)
    pltpu.sync_copy(dstR.at[pl.ds(wid * KCH, KCH)], dst2)

    def chunk(k, _):
        def grp(g, _):
            sl = pl.ds(g * L, L)
            s16 = src2[k, sl]
            d16 = dst2[k, sl]
            a = plsc.load_gather(asv, [s16]) + plsc.load_gather(adv, [d16])
            a = jnp.where(a >= 0.0, a, 0.2 * a)
            e = jnp.exp(a)
            ex2[k, sl] = e
            plsc.addupdate_scatter(den_v, [d16], e)
            return 0
        lax.fori_loop(0, 128 // L, grp, 0)
        return 0
    lax.fori_loop(0, KCH, chunk, 0)

    pltpu.sync_copy(ex2, exR.at[pl.ds(wid * KCH, KCH)])

    pltpu.sync_copy(den_v, stage.at[s])
    plsc.subcore_barrier()

    def zacc(i, _):
        accv[pl.ds(i * L, L)] = z16
        return 0
    lax.fori_loop(0, RPT // L, zacc, 0)

    def comb(t, _):
        pltpu.sync_copy(stage.at[t, pl.ds(s * RPT, RPT)], tmp_v)

        def addg(g, _):
            sl = pl.ds(g * L, L)
            accv[sl] = accv[sl] + tmp_v[sl]
            return 0
        lax.fori_loop(0, RPT // L, addg, 0)
        return 0
    lax.fori_loop(0, NS, comb, 0)

    pltpu.sync_copy(accv, denp.at[c, pl.ds(s * RPT, RPT)])


@functools.partial(
    pl.kernel,
    out_type=jax.ShapeDtypeStruct((NC, N_PAD, F), jnp.float32),
    mesh=_MESH,
    compiler_params=pltpu.CompilerParams(needs_layout_passes=False,
                                         use_tc_tiling_on_sc=False),
    scratch_types=[
        pltpu.VMEM((KC4, 64), jnp.int32),
        pltpu.VMEM((2, 64), jnp.int32),
        pltpu.VMEM((2, 64), jnp.float32),
        pltpu.VMEM((2, 64), jnp.float32),
        pltpu.VMEM((2, 64, F), jnp.bfloat16),
        pltpu.VMEM((2, 64, F), jnp.float32),
        pltpu.VMEM((RPT,), jnp.float32),
        pltpu.VMEM((RPT,), jnp.float32),
        pltpu.VMEM((64, F), jnp.float32),
        pltpu.SemaphoreType.DMA((2,)),
        pltpu.SemaphoreType.DMA((2,)),
        pltpu.SemaphoreType.DMA((2,)),
        pltpu.SemaphoreType.DMA((2,)),
        pltpu.VMEM_SHARED((N_PAD,), jnp.float32),
        pltpu.VMEM_SHARED((N_PAD, F), jnp.float32),
    ],
)
def _edge_aggregate(hbf, srcR4, dstR4, exR4, denp, outP,
                    dst2, sbuf, xbuf, denb, rows_bf, rows_f, p0b, p1b, zrow,
                    isem, dsem, gsem, ssem, den_sh, acc):
    c = lax.axis_index("c")
    s = lax.axis_index("s")
    wid = c * NS + s
    zf = jnp.zeros((L,), jnp.float32)

    pltpu.sync_copy(denp.at[0, pl.ds(s * RPT, RPT)], p0b)
    pltpu.sync_copy(denp.at[1, pl.ds(s * RPT, RPT)], p1b)

    def drecip(i, _):
        sl = pl.ds(i * L, L)
        p0b[sl] = 1.0 / (p0b[sl] + p1b[sl] + 1e-16)
        return 0
    lax.fori_loop(0, RPT // L, drecip, 0)
    pltpu.sync_copy(p0b, den_sh.at[pl.ds(s * RPT, RPT)])

    def zr(i, _):
        for j in range(F // L):
            zrow[i, pl.ds(j * L, L)] = zf
        return 0
    lax.fori_loop(0, 64, zr, 0)
    for b in range(RPT // 64):
        pltpu.sync_copy(zrow, acc.at[pl.ds(s * RPT + b * 64, 64)])

    pltpu.sync_copy(dstR4.at[pl.ds(wid * KC4, KC4)], dst2)
    plsc.subcore_barrier()

    base = wid * KC4

    def issue_idx(k, b):
        pltpu.async_copy(srcR4.at[base + k], sbuf.at[b], isem.at[b])
        pltpu.async_copy(exR4.at[base + k], xbuf.at[b], isem.at[b])

    def wait_idx(k, b):
        pltpu.make_async_copy(srcR4.at[base + k], sbuf.at[b], isem.at[b]).wait()
        pltpu.make_async_copy(exR4.at[base + k], xbuf.at[b], isem.at[b]).wait()

    def issue_den(k, b):
        pltpu.async_copy(den_sh.at[dst2.at[k]], denb.at[b], dsem.at[b])

    issue_idx(0, 0)
    issue_den(0, 0)
    issue_den(1, 1)
    wait_idx(0, 0)
    pltpu.async_copy(hbf.at[sbuf.at[0]], rows_bf.at[0], gsem.at[0])
    issue_idx(1, 1)

    def chunk(k, _):
        b = lax.rem(k, 2)
        nb2 = lax.rem(k + 1, 2)

        @pl.when(k + 1 < KC4)
        def _():
            wait_idx(k + 1, nb2)
            pltpu.async_copy(hbf.at[sbuf.at[nb2]], rows_bf.at[nb2],
                             gsem.at[nb2])

        pltpu.make_async_copy(hbf.at[sbuf.at[b]], rows_bf.at[b],
                              gsem.at[b]).wait()
        pltpu.make_async_copy(den_sh.at[dst2.at[k]], denb.at[b],
                              dsem.at[b]).wait()

        @pl.when(k >= 2)
        def _():
            pltpu.make_async_copy(rows_f.at[b], acc.at[dst2.at[k - 2]],
                                  ssem.at[b]).wait()

        def scale(g, _):
            sl = pl.ds(g * L, L)
            cvec = xbuf[b, sl] * denb[b, sl]
            for li in range(L):
                r = g * L + li
                cfi = cvec[li]
                for j in range(F // 32):
                    ab = rows_bf[b, r, pl.ds(j * 32, 32)]
                    alo, ahi = plsc.unpack(
                        ab, format=plsc.PackFormat.INTERLEAVED,
                        preferred_element_type=jnp.float32)
                    rows_f[b, r, pl.ds(j * 32, L)] = alo * cfi
                    rows_f[b, r, pl.ds(j * 32 + L, L)] = ahi * cfi
            return 0
        lax.fori_loop(0, 64 // L, scale, 0)

        pltpu.async_copy(rows_f.at[b], acc.at[dst2.at[k]], ssem.at[b],
                         add=True)

        @pl.when(k + 2 < KC4)
        def _():
            issue_idx(k + 2, b)
            issue_den(k + 2, b)
        return 0
    lax.fori_loop(0, KC4, chunk, 0)

    for b in range(2):
        pltpu.make_async_copy(rows_f.at[b], acc.at[dst2.at[0]],
                              ssem.at[b]).wait()

    plsc.subcore_barrier()
    pltpu.sync_copy(acc.at[pl.ds(s * RPT, RPT)],
                    outP.at[c, pl.ds(s * RPT, RPT)])


def kernel(x, edge_index, batch, W1, att_src1, att_dst1, W2, att_src2, att_dst2):
    src = edge_index[0]
    dst = edge_index[1]
    pad = E_PAD - E
    srcp = jnp.concatenate([src, jnp.zeros((pad,), jnp.int32)])
    dstp = jnp.concatenate([dst, jnp.full((pad,), N, jnp.int32)])
    srcR = srcp.reshape(ER, 128)
    dstR = dstp.reshape(ER, 128)
    srcR4 = srcp.reshape(ER4, 64)
    dstR4 = dstp.reshape(ER4, 64)
    a21 = jnp.stack([att_src1, att_dst1], axis=1)
    a22 = jnp.stack([att_src2, att_dst2], axis=1)

    h1, aout1 = _mm_alpha(x, W1, a21)
    ex1, denp1 = _edge_softmax(srcR, dstR, aout1[:, 0], aout1[:, 1])
    P1 = _edge_aggregate(h1, srcR4, dstR4, ex1.reshape(ER4, 64), denp1)

    h2, aout2 = _mm_relu_alpha(P1, W2, a22)
    ex2, denp2 = _edge_softmax(srcR, dstR, aout2[:N, 0], aout2[:N, 1])
    P2 = _edge_aggregate(h2, srcR4, dstR4, ex2.reshape(ER4, 64), denp2)

    return _combine(P2)


# speedup vs baseline: 1.3743x; 1.3743x over previous
import functools

import jax
import jax.numpy as jnp
from jax import lax
from jax.experimental import pallas as pl
from jax.experimental.pallas import tpu as pltpu
from jax.experimental.pallas import tpu_sc as plsc

N = 10000
F = 128
E = 320000
NC = 2
NS = 16
NW = NC * NS
L = 16
N_PAD = 10240
RPT = N_PAD // NS
EPT = 10240
E_PAD = EPT * NW
KCH = EPT // 128
ER = E_PAD // 128
ER4 = E_PAD // 64
KC4 = EPT // 64


def _mm_alpha_body(x_ref, w_ref, a2_ref, h_ref, aout_ref):
    h = jnp.dot(x_ref[...], w_ref[...], preferred_element_type=jnp.float32)
    h_ref[...] = h
    aout_ref[...] = jnp.dot(h, a2_ref[...], preferred_element_type=jnp.float32)


def _mm_alpha(x, w, a2, blk=2000):
    n = x.shape[0]
    return pl.pallas_call(
        _mm_alpha_body,
        grid=(n // blk,),
        in_specs=[pl.BlockSpec((blk, F), lambda i: (i, 0)),
                  pl.BlockSpec((F, F), lambda i: (0, 0)),
                  pl.BlockSpec((F, 2), lambda i: (0, 0))],
        out_specs=[pl.BlockSpec((blk, F), lambda i: (i, 0)),
                   pl.BlockSpec((blk, 2), lambda i: (i, 0))],
        out_shape=[jax.ShapeDtypeStruct((n, F), jnp.float32),
                   jax.ShapeDtypeStruct((n, 2), jnp.float32)],
    )(x, w, a2)


def _mm_relu_alpha_body(p_ref, w_ref, a2_ref, h_ref, aout_ref):
    g = jnp.maximum(p_ref[0] + p_ref[1], 0.0)
    h = jnp.dot(g, w_ref[...], preferred_element_type=jnp.float32)
    h_ref[...] = h
    aout_ref[...] = jnp.dot(h, a2_ref[...], preferred_element_type=jnp.float32)


def _mm_relu_alpha(p, w, a2, blk=2048):
    n = p.shape[1]
    return pl.pallas_call(
        _mm_relu_alpha_body,
        grid=(n // blk,),
        in_specs=[pl.BlockSpec((2, blk, F), lambda i: (0, i, 0)),
                  pl.BlockSpec((F, F), lambda i: (0, 0)),
                  pl.BlockSpec((F, 2), lambda i: (0, 0))],
        out_specs=[pl.BlockSpec((blk, F), lambda i: (i, 0)),
                   pl.BlockSpec((blk, 2), lambda i: (i, 0))],
        out_shape=[jax.ShapeDtypeStruct((n, F), jnp.float32),
                   jax.ShapeDtypeStruct((n, 2), jnp.float32)],
    )(p, w, a2)


def _combine_body(p_ref, o_ref):
    o_ref[...] = p_ref[0] + p_ref[1]


def _combine(p, blk=2000):
    return pl.pallas_call(
        _combine_body,
        grid=(N // blk,),
        in_specs=[pl.BlockSpec((2, blk, F), lambda i: (0, i, 0))],
        out_specs=pl.BlockSpec((blk, F), lambda i: (i, 0)),
        out_shape=jax.ShapeDtypeStruct((N, F), jnp.float32),
    )(p)


_MESH = plsc.VectorSubcoreMesh(core_axis_name="c", subcore_axis_name="s")


@functools.partial(
    pl.kernel,
    out_type=[jax.ShapeDtypeStruct((ER, 128), jnp.float32),
              jax.ShapeDtypeStruct((NC, N_PAD), jnp.float32)],
    mesh=_MESH,
    compiler_params=pltpu.CompilerParams(needs_layout_passes=False),
    scratch_types=[
        pltpu.VMEM((N_PAD,), jnp.float32),
        pltpu.VMEM((N_PAD,), jnp.float32),
        pltpu.VMEM((KCH, 128), jnp.int32),
        pltpu.VMEM((KCH, 128), jnp.int32),
        pltpu.VMEM((KCH, 128), jnp.float32),
        pltpu.VMEM((N_PAD,), jnp.float32),
        pltpu.VMEM((RPT,), jnp.float32),
        pltpu.VMEM((RPT,), jnp.float32),
        pltpu.VMEM_SHARED((NS, N_PAD), jnp.float32),
    ],
)
def _edge_softmax(srcR, dstR, asrc, adst, exR, denp,
                  asv, adv, src2, dst2, ex2, den_v, tmp_v, accv, stage):
    c = lax.axis_index("c")
    s = lax.axis_index("s")
    wid = c * NS + s
    z16 = jnp.zeros((L,), jnp.float32)

    def zset(i, _):
        asv[pl.ds(i * L, L)] = z16
        adv[pl.ds(i * L, L)] = z16
        den_v[pl.ds(i * L, L)] = z16
        return 0
    lax.fori_loop(0, N_PAD // L, zset, 0)

    pltpu.sync_copy(asrc, asv.at[pl.ds(0, N)])
    pltpu.sync_copy(adst, adv.at[pl.ds(0, N)])
    pltpu.sync_copy(srcR.at[pl.ds(wid * KCH, KCH)], src2)
    pltpu.sync_copy(dstR.at[pl.ds(wid * KCH, KCH)], dst2)

    def chunk(k, _):
        def grp(g, _):
            sl = pl.ds(g * L, L)
            s16 = src2[k, sl]
            d16 = dst2[k, sl]
            a = plsc.load_gather(asv, [s16]) + plsc.load_gather(adv, [d16])
            a = jnp.where(a >= 0.0, a, 0.2 * a)
            e = jnp.exp(a)
            ex2[k, sl] = e
            plsc.addupdate_scatter(den_v, [d16], e)
            return 0
        lax.fori_loop(0, 128 // L, grp, 0)
        return 0
    lax.fori_loop(0, KCH, chunk, 0)

    pltpu.sync_copy(ex2, exR.at[pl.ds(wid * KCH, KCH)])

    pltpu.sync_copy(den_v, stage.at[s])
    plsc.subcore_barrier()

    def zacc(i, _):
        accv[pl.ds(i * L, L)] = z16
        return 0
    lax.fori_loop(0, RPT // L, zacc, 0)

    def comb(t, _):
        pltpu.sync_copy(stage.at[t, pl.ds(s * RPT, RPT)], tmp_v)

        def addg(g, _):
            sl = pl.ds(g * L, L)
            accv[sl] = accv[sl] + tmp_v[sl]
            return 0
        lax.fori_loop(0, RPT // L, addg, 0)
        return 0
    lax.fori_loop(0, NS, comb, 0)

    pltpu.sync_copy(accv, denp.at[c, pl.ds(s * RPT, RPT)])


@functools.partial(
    pl.kernel,
    out_type=jax.ShapeDtypeStruct((NC, N_PAD, F), jnp.float32),
    mesh=_MESH,
    compiler_params=pltpu.CompilerParams(needs_layout_passes=False,
                                         use_tc_tiling_on_sc=False),
    scratch_types=[
        pltpu.VMEM((KC4, 64), jnp.int32),
        pltpu.VMEM((2, 64), jnp.int32),
        pltpu.VMEM((2, 64), jnp.float32),
        pltpu.VMEM((2, 64), jnp.float32),
        pltpu.VMEM((2, 64, F), jnp.bfloat16),
        pltpu.VMEM((2, 64, F), jnp.float32),
        pltpu.VMEM((RPT,), jnp.float32),
        pltpu.VMEM((RPT,), jnp.float32),
        pltpu.VMEM((64, F), jnp.float32),
        pltpu.SemaphoreType.DMA((2,)),
        pltpu.SemaphoreType.DMA((2,)),
        pltpu.SemaphoreType.DMA((2,)),
        pltpu.SemaphoreType.DMA((2,)),
        pltpu.VMEM_SHARED((N_PAD,), jnp.float32),
        pltpu.VMEM_SHARED((N_PAD, F), jnp.float32),
    ],
)
def _edge_aggregate(hbf, srcR4, dstR4, exR4, denp, outP,
                    dst2, sbuf, xbuf, denb, rows_bf, rows_f, p0b, p1b, zrow,
                    isem, dsem, gsem, ssem, den_sh, acc):
    c = lax.axis_index("c")
    s = lax.axis_index("s")
    wid = c * NS + s
    zf = jnp.zeros((L,), jnp.float32)

    pltpu.sync_copy(denp.at[0, pl.ds(s * RPT, RPT)], p0b)
    pltpu.sync_copy(denp.at[1, pl.ds(s * RPT, RPT)], p1b)

    def drecip(i, _):
        sl = pl.ds(i * L, L)
        p0b[sl] = 1.0 / (p0b[sl] + p1b[sl] + 1e-16)
        return 0
    lax.fori_loop(0, RPT // L, drecip, 0)
    pltpu.sync_copy(p0b, den_sh.at[pl.ds(s * RPT, RPT)])

    def zr(i, _):
        for j in range(F // L):
            zrow[i, pl.ds(j * L, L)] = zf
        return 0
    lax.fori_loop(0, 64, zr, 0)
    for b in range(RPT // 64):
        pltpu.sync_copy(zrow, acc.at[pl.ds(s * RPT + b * 64, 64)])

    pltpu.sync_copy(dstR4.at[pl.ds(wid * KC4, KC4)], dst2)
    plsc.subcore_barrier()

    base = wid * KC4

    def issue_idx(k, b):
        pltpu.async_copy(srcR4.at[base + k], sbuf.at[b], isem.at[b])
        pltpu.async_copy(exR4.at[base + k], xbuf.at[b], isem.at[b])

    def wait_idx(k, b):
        pltpu.make_async_copy(srcR4.at[base + k], sbuf.at[b], isem.at[b]).wait()
        pltpu.make_async_copy(exR4.at[base + k], xbuf.at[b], isem.at[b]).wait()

    def issue_den(k, b):
        pltpu.async_copy(den_sh.at[dst2.at[k]], denb.at[b], dsem.at[b])

    issue_idx(0, 0)
    issue_den(0, 0)
    issue_den(1, 1)
    wait_idx(0, 0)
    pltpu.async_copy(hbf.at[sbuf.at[0]], rows_bf.at[0], gsem.at[0])
    issue_idx(1, 1)

    def chunk(k, _):
        b = lax.rem(k, 2)
        nb2 = lax.rem(k + 1, 2)

        @pl.when(k + 1 < KC4)
        def _():
            wait_idx(k + 1, nb2)
            pltpu.async_copy(hbf.at[sbuf.at[nb2]], rows_bf.at[nb2],
                             gsem.at[nb2])

        pltpu.make_async_copy(hbf.at[sbuf.at[b]], rows_bf.at[b],
                              gsem.at[b]).wait()
        pltpu.make_async_copy(den_sh.at[dst2.at[k]], denb.at[b],
                              dsem.at[b]).wait()

        @pl.when(k >= 2)
        def _():
            pltpu.make_async_copy(rows_f.at[b], acc.at[dst2.at[k - 2]],
                                  ssem.at[b]).wait()

        def scale(g, _):
            sl = pl.ds(g * L, L)
            cvec = xbuf[b, sl] * denb[b, sl]
            for li in range(L):
                r = g * L + li
                cfi = cvec[li]
                for j in range(F // 32):
                    ab = rows_bf[b, r, pl.ds(j * 32, 32)]
                    alo, ahi = plsc.unpack(
                        ab, format=plsc.PackFormat.INTERLEAVED,
                        preferred_element_type=jnp.float32)
                    rows_f[b, r, pl.ds(j * 32, L)] = alo * cfi
                    rows_f[b, r, pl.ds(j * 32 + L, L)] = ahi * cfi
            return 0
        lax.fori_loop(0, 64 // L, scale, 0)

        pltpu.async_copy(rows_f.at[b], acc.at[dst2.at[k]], ssem.at[b],
                         add=True)

        @pl.when(k + 2 < KC4)
        def _():
            issue_idx(k + 2, b)
            issue_den(k + 2, b)
        return 0
    lax.fori_loop(0, KC4, chunk, 0)

    for b in range(2):
        pltpu.make_async_copy(rows_f.at[b], acc.at[dst2.at[0]],
                              ssem.at[b]).wait()

    plsc.subcore_barrier()
    pltpu.sync_copy(acc.at[pl.ds(s * RPT, RPT)],
                    outP.at[c, pl.ds(s * RPT, RPT)])


def _perm_bf16(h):
    n = h.shape[0]
    hp = h.reshape(n, 4, 2, 16).transpose(0, 1, 3, 2).reshape(n, F)
    return hp.astype(jnp.bfloat16)


def kernel(x, edge_index, batch, W1, att_src1, att_dst1, W2, att_src2, att_dst2):
    src = edge_index[0]
    dst = edge_index[1]
    pad = E_PAD - E
    srcp = jnp.concatenate([src, jnp.zeros((pad,), jnp.int32)])
    dstp = jnp.concatenate([dst, jnp.full((pad,), N, jnp.int32)])
    srcR = srcp.reshape(ER, 128)
    dstR = dstp.reshape(ER, 128)
    srcR4 = srcp.reshape(ER4, 64)
    dstR4 = dstp.reshape(ER4, 64)
    a21 = jnp.stack([att_src1, att_dst1], axis=1)
    a22 = jnp.stack([att_src2, att_dst2], axis=1)

    h1, aout1 = _mm_alpha(x, W1, a21)
    ex1, denp1 = _edge_softmax(srcR, dstR, aout1[:, 0], aout1[:, 1])
    P1 = _edge_aggregate(_perm_bf16(h1), srcR4, dstR4, ex1.reshape(ER4, 64), denp1)

    h2, aout2 = _mm_relu_alpha(P1, W2, a22)
    ex2, denp2 = _edge_softmax(srcR, dstR, aout2[:N, 0], aout2[:N, 1])
    P2 = _edge_aggregate(_perm_bf16(h2), srcR4, dstR4, ex2.reshape(ER4, 64), denp2)

    return _combine(P2)


# speedup vs baseline: 1.5098x; 1.0986x over previous
import functools

import jax
import jax.numpy as jnp
from jax import lax
from jax.experimental import pallas as pl
from jax.experimental.pallas import tpu as pltpu
from jax.experimental.pallas import tpu_sc as plsc

N = 10000
F = 128
E = 320000
NC = 2
NS = 16
NW = NC * NS
L = 16
N_PAD = 10240
RPT = N_PAD // NS
EPT = 10240
E_PAD = EPT * NW
KCH = EPT // 128
ER = E_PAD // 128
ER4 = E_PAD // 64
KC4 = EPT // 64


def _mm_alpha_body(x_ref, w_ref, a2_ref, h_ref, aout_ref):
    h = jnp.dot(x_ref[...], w_ref[...], preferred_element_type=jnp.float32)
    h_ref[...] = h
    aout_ref[...] = jnp.dot(h, a2_ref[...], preferred_element_type=jnp.float32)


def _mm_alpha(x, w, a2, blk=2000):
    n = x.shape[0]
    return pl.pallas_call(
        _mm_alpha_body,
        grid=(n // blk,),
        in_specs=[pl.BlockSpec((blk, F), lambda i: (i, 0)),
                  pl.BlockSpec((F, F), lambda i: (0, 0)),
                  pl.BlockSpec((F, 2), lambda i: (0, 0))],
        out_specs=[pl.BlockSpec((blk, F), lambda i: (i, 0)),
                   pl.BlockSpec((blk, 2), lambda i: (i, 0))],
        out_shape=[jax.ShapeDtypeStruct((n, F), jnp.float32),
                   jax.ShapeDtypeStruct((n, 2), jnp.float32)],
    )(x, w, a2)


def _mm_relu_alpha_body(p_ref, w_ref, a2_ref, h_ref, aout_ref):
    g = jnp.maximum(p_ref[0] + p_ref[1], 0.0)
    h = jnp.dot(g, w_ref[...], preferred_element_type=jnp.float32)
    h_ref[...] = h
    aout_ref[...] = jnp.dot(h, a2_ref[...], preferred_element_type=jnp.float32)


def _mm_relu_alpha(p, w, a2, blk=2048):
    n = p.shape[1]
    return pl.pallas_call(
        _mm_relu_alpha_body,
        grid=(n // blk,),
        in_specs=[pl.BlockSpec((2, blk, F), lambda i: (0, i, 0)),
                  pl.BlockSpec((F, F), lambda i: (0, 0)),
                  pl.BlockSpec((F, 2), lambda i: (0, 0))],
        out_specs=[pl.BlockSpec((blk, F), lambda i: (i, 0)),
                   pl.BlockSpec((blk, 2), lambda i: (i, 0))],
        out_shape=[jax.ShapeDtypeStruct((n, F), jnp.float32),
                   jax.ShapeDtypeStruct((n, 2), jnp.float32)],
    )(p, w, a2)


def _combine_body(p_ref, o_ref):
    o_ref[...] = p_ref[0] + p_ref[1]


def _combine(p, blk=2000):
    return pl.pallas_call(
        _combine_body,
        grid=(N // blk,),
        in_specs=[pl.BlockSpec((2, blk, F), lambda i: (0, i, 0))],
        out_specs=pl.BlockSpec((blk, F), lambda i: (i, 0)),
        out_shape=jax.ShapeDtypeStruct((N, F), jnp.float32),
    )(p)


_MESH = plsc.VectorSubcoreMesh(core_axis_name="c", subcore_axis_name="s")


@functools.partial(
    pl.kernel,
    out_type=[jax.ShapeDtypeStruct((ER, 128), jnp.float32),
              jax.ShapeDtypeStruct((NC, N_PAD), jnp.float32)],
    mesh=_MESH,
    compiler_params=pltpu.CompilerParams(needs_layout_passes=False),
    scratch_types=[
        pltpu.VMEM((N_PAD,), jnp.float32),
        pltpu.VMEM((N_PAD,), jnp.float32),
        pltpu.VMEM((KCH, 128), jnp.int32),
        pltpu.VMEM((KCH, 128), jnp.int32),
        pltpu.VMEM((KCH, 128), jnp.float32),
        pltpu.VMEM((N_PAD,), jnp.float32),
        pltpu.VMEM((RPT,), jnp.float32),
        pltpu.VMEM((RPT,), jnp.float32),
        pltpu.VMEM_SHARED((NS, N_PAD), jnp.float32),
    ],
)
def _edge_softmax(srcR, dstR, asrc, adst, exR, denp,
                  asv, adv, src2, dst2, ex2, den_v, tmp_v, accv, stage):
    c = lax.axis_index("c")
    s = lax.axis_index("s")
    wid = c * NS + s
    z16 = jnp.zeros((L,), jnp.float32)

    def zset(i, _):
        asv[pl.ds(i * L, L)] = z16
        adv[pl.ds(i * L, L)] = z16
        den_v[pl.ds(i * L, L)] = z16
        return 0
    lax.fori_loop(0, N_PAD // L, zset, 0)

    pltpu.sync_copy(asrc, asv.at[pl.ds(0, N)])
    pltpu.sync_copy(adst, adv.at[pl.ds(0, N)])
    pltpu.sync_copy(srcR.at[pl.ds(wid * KCH, KCH)], src2)
    pltpu.sync_copy(dstR.at[pl.ds(wid * KCH, KCH)], dst2)

    def chunk(k, _):
        def grp(g, _):
            sl = pl.ds(g * L, L)
            s16 = src2[k, sl]
            d16 = dst2[k, sl]
            a = plsc.load_gather(asv, [s16]) + plsc.load_gather(adv, [d16])
            a = jnp.where(a >= 0.0, a, 0.2 * a)
            e = jnp.exp(a)
            ex2[k, sl] = e
            plsc.addupdate_scatter(den_v, [d16], e)
            return 0
        lax.fori_loop(0, 128 // L, grp, 0)
        return 0
    lax.fori_loop(0, KCH, chunk, 0)

    pltpu.sync_copy(ex2, exR.at[pl.ds(wid * KCH, KCH)])

    pltpu.sync_copy(den_v, stage.at[s])
    plsc.subcore_barrier()

    def zacc(i, _):
        accv[pl.ds(i * L, L)] = z16
        return 0
    lax.fori_loop(0, RPT // L, zacc, 0)

    def comb(t, _):
        pltpu.sync_copy(stage.at[t, pl.ds(s * RPT, RPT)], tmp_v)

        def addg(g, _):
            sl = pl.ds(g * L, L)
            accv[sl] = accv[sl] + tmp_v[sl]
            return 0
        lax.fori_loop(0, RPT // L, addg, 0)
        return 0
    lax.fori_loop(0, NS, comb, 0)

    pltpu.sync_copy(accv, denp.at[c, pl.ds(s * RPT, RPT)])


@functools.partial(
    pl.kernel,
    out_type=jax.ShapeDtypeStruct((NC, N_PAD, F), jnp.float32),
    mesh=_MESH,
    compiler_params=pltpu.CompilerParams(needs_layout_passes=False,
                                         use_tc_tiling_on_sc=False),
    scratch_types=[
        pltpu.VMEM((KC4, 64), jnp.int32),
        pltpu.VMEM((2, 128), jnp.int32),
        pltpu.VMEM((2, 128), jnp.float32),
        pltpu.VMEM((4, 64), jnp.float32),
        pltpu.VMEM((2, 128, F), jnp.bfloat16),
        pltpu.VMEM((2, 64, F), jnp.float32),
        pltpu.VMEM((RPT,), jnp.float32),
        pltpu.VMEM((RPT,), jnp.float32),
        pltpu.VMEM((16, F), jnp.float32),
        pltpu.SemaphoreType.DMA((2,)),
        pltpu.SemaphoreType.DMA((4,)),
        pltpu.SemaphoreType.DMA((2,)),
        pltpu.SemaphoreType.DMA((2,)),
        pltpu.VMEM_SHARED((N_PAD,), jnp.float32),
        pltpu.VMEM_SHARED((N_PAD, F), jnp.float32),
    ],
)
def _edge_aggregate(hbf, srcR, dstR4, exR, denp, outP,
                    dst2, sbuf, xbuf, denb, rows_bf, rows_f, p0b, p1b, zrow,
                    isem, dsem, gsem, ssem, den_sh, acc):
    c = lax.axis_index("c")
    s = lax.axis_index("s")
    wid = c * NS + s
    zf = jnp.zeros((L,), jnp.float32)

    pltpu.sync_copy(denp.at[0, pl.ds(s * RPT, RPT)], p0b)
    pltpu.sync_copy(denp.at[1, pl.ds(s * RPT, RPT)], p1b)

    def drecip(i, _):
        sl = pl.ds(i * L, L)
        p0b[sl] = 1.0 / (p0b[sl] + p1b[sl] + 1e-16)
        return 0
    lax.fori_loop(0, RPT // L, drecip, 0)
    pltpu.sync_copy(p0b, den_sh.at[pl.ds(s * RPT, RPT)])

    def zr(i, _):
        for j in range(F // L):
            zrow[i, pl.ds(j * L, L)] = zf
        return 0
    lax.fori_loop(0, 16, zr, 0)
    for b in range(RPT // 16):
        pltpu.sync_copy(zrow, acc.at[pl.ds(s * RPT + b * 16, 16)])

    pltpu.sync_copy(dstR4.at[pl.ds(wid * KC4, KC4)], dst2)
    plsc.subcore_barrier()

    base = wid * KCH

    def issue_idx(gk, b):
        pltpu.async_copy(srcR.at[base + gk], sbuf.at[b], isem.at[b])
        pltpu.async_copy(exR.at[base + gk], xbuf.at[b], isem.at[b])

    def wait_idx(gk, b):
        pltpu.make_async_copy(srcR.at[base + gk], sbuf.at[b], isem.at[b]).wait()
        pltpu.make_async_copy(exR.at[base + gk], xbuf.at[b], isem.at[b]).wait()

    base4 = wid * KC4

    def issue_den(k64, db):
        pltpu.async_copy(den_sh.at[dst2.at[base4 - base4 + k64]],
                         denb.at[db], dsem.at[db])

    issue_idx(0, 0)
    for d in range(4):
        issue_den(d, d)
    wait_idx(0, 0)
    pltpu.async_copy(hbf.at[sbuf.at[0]], rows_bf.at[0], gsem.at[0])
    issue_idx(1, 1)

    def chunk(gk, _):
        b = lax.rem(gk, 2)
        nb2 = lax.rem(gk + 1, 2)

        @pl.when(gk + 1 < KCH)
        def _():
            wait_idx(gk + 1, nb2)
            pltpu.async_copy(hbf.at[sbuf.at[nb2]], rows_bf.at[nb2],
                             gsem.at(nb2) if False else gsem.at[nb2])

        pltpu.make_async_copy(hbf.at[sbuf.at[b]], rows_bf.at[b],
                              gsem.at[b]).wait()

        for half in range(2):
            k64 = 2 * gk + half
            db = lax.rem(k64, 4)
            pltpu.make_async_copy(den_sh.at[dst2.at[k64]], denb.at[db],
                                  dsem.at[db]).wait()

            @pl.when(k64 >= 2)
            def _():
                pltpu.make_async_copy(rows_f.at[half], acc.at[dst2.at[k64 - 2]],
                                      ssem.at[half]).wait()

            def scale(g, _):
                sl64 = pl.ds(g * L, L)
                cvec = xbuf[b, pl.ds(half * 64 + g * L, L)] * denb[db, sl64]
                for li in range(L):
                    rbf = half * 64 + g * L + li
                    rf = g * L + li
                    cfi = cvec[li]
                    for j in range(F // 32):
                        ab = rows_bf[b, rbf, pl.ds(j * 32, 32)]
                        alo, ahi = plsc.unpack(
                            ab, format=plsc.PackFormat.INTERLEAVED,
                            preferred_element_type=jnp.float32)
                        rows_f[half, rf, pl.ds(j * 32, L)] = alo * cfi
                        rows_f[half, rf, pl.ds(j * 32 + L, L)] = ahi * cfi
                return 0
            lax.fori_loop(0, 64 // L, scale, 0)

            pltpu.async_copy(rows_f.at[half], acc.at[dst2.at[k64]],
                             ssem.at[half], add=True)

            @pl.when(k64 + 4 < KC4)
            def _():
                issue_den(k64 + 4, db)

        @pl.when(gk + 2 < KCH)
        def _():
            issue_idx(gk + 2, b)
        return 0
    lax.fori_loop(0, KCH, chunk, 0)

    for b in range(2):
        pltpu.make_async_copy(rows_f.at[b], acc.at[dst2.at[0]],
                              ssem.at[b]).wait()

    plsc.subcore_barrier()
    pltpu.sync_copy(acc.at[pl.ds(s * RPT, RPT)],
                    outP.at[c, pl.ds(s * RPT, RPT)])


def _perm_bf16(h):
    n = h.shape[0]
    hp = h.reshape(n, 4, 2, 16).transpose(0, 1, 3, 2).reshape(n, F)
    return hp.astype(jnp.bfloat16)


def kernel(x, edge_index, batch, W1, att_src1, att_dst1, W2, att_src2, att_dst2):
    src = edge_index[0]
    dst = edge_index[1]
    pad = E_PAD - E
    srcp = jnp.concatenate([src, jnp.zeros((pad,), jnp.int32)])
    dstp = jnp.concatenate([dst, jnp.full((pad,), N, jnp.int32)])
    srcR = srcp.reshape(ER, 128)
    dstR = dstp.reshape(ER, 128)
    srcR4 = srcp.reshape(ER4, 64)
    dstR4 = dstp.reshape(ER4, 64)
    a21 = jnp.stack([att_src1, att_dst1], axis=1)
    a22 = jnp.stack([att_src2, att_dst2], axis=1)

    h1, aout1 = _mm_alpha(x, W1, a21)
    ex1, denp1 = _edge_softmax(srcR, dstR, aout1[:, 0], aout1[:, 1])
    P1 = _edge_aggregate(_perm_bf16(h1), srcR, dstR4, ex1, denp1)

    h2, aout2 = _mm_relu_alpha(P1, W2, a22)
    ex2, denp2 = _edge_softmax(srcR, dstR, aout2[:N, 0], aout2[:N, 1])
    P2 = _edge_aggregate(_perm_bf16(h2), srcR, dstR4, ex2, denp2)

    return _combine(P2)
